# baseline (speedup 1.0000x reference)
import functools

import jax
import jax.numpy as jnp
from jax import lax
from jax.experimental import pallas as pl
from jax.experimental.pallas import tpu as pltpu
from jax.experimental.pallas import tpu_sc as plsc

_D = 64
_K = 20
_L = 16
_NC, _NS = 2, 16
_NW = _NC * _NS


def _sc_scores(center, context, negs, comb, *, B, C):
    PW = B // _NW
    NCH = PW // C
    mesh = plsc.VectorSubcoreMesh(core_axis_name="c", subcore_axis_name="s")

    @functools.partial(
        pl.kernel,
        out_type=jax.ShapeDtypeStruct(((_K + 1) * B,), jnp.float32),
        mesh=mesh,
        compiler_params=pltpu.CompilerParams(needs_layout_passes=False),
        scratch_types=[
            pltpu.VMEM((PW,), jnp.int32),
            pltpu.VMEM((PW,), jnp.int32),
            pltpu.VMEM((_K * PW,), jnp.int32),
            pltpu.VMEM((C, 2 * _D), jnp.float32),
            pltpu.VMEM((C, 2 * _D), jnp.float32),
            pltpu.VMEM((_K * C, 2 * _D), jnp.float32),
            pltpu.VMEM(((_K + 1) * PW,), jnp.float32),
            pltpu.SemaphoreType.DMA,
        ],
    )
    def k(center_hbm, context_hbm, negsT_hbm, comb_hbm, out_hbm,
          cidx, tidx, nidx, cbuf, tbuf, nbuf, scores, sem):
        wid = lax.axis_index("s") * _NC + lax.axis_index("c")
        wbase = wid * PW
        idx_descs = [
            pltpu.async_copy(center_hbm.at[pl.ds(wbase, PW)], cidx, sem),
            pltpu.async_copy(context_hbm.at[pl.ds(wbase, PW)], tidx, sem),
        ]
        for j in range(_K):
            idx_descs.append(
                pltpu.async_copy(negsT_hbm.at[j, pl.ds(wbase, PW)],
                                 nidx.at[pl.ds(j * PW, PW)], sem))
        for dsc in idx_descs:
            dsc.wait()

        for ch in range(NCH):
            off = ch * C
            descs = [
                pltpu.async_copy(comb_hbm.at[cidx.at[pl.ds(off, C)]],
                                 cbuf, sem),
                pltpu.async_copy(comb_hbm.at[tidx.at[pl.ds(off, C)]],
                                 tbuf, sem),
            ]
            for j in range(_K):
                descs.append(
                    pltpu.async_copy(
                        comb_hbm.at[nidx.at[pl.ds(j * PW + off, C)]],
                        nbuf.at[pl.ds(j * C, C), :], sem))
            for dsc in descs:
                dsc.wait()

            lanes = lax.iota(jnp.int32, _L)
            for g in range(C // _L):
                rows = lanes + (g * _L)

                def dbody(i, carry, rows=rows):
                    dvec = carry[0]
                    accs = carry[1:]
                    dhi = dvec + _D
                    vc = plsc.load_gather(cbuf, [rows, dvec])
                    vt = plsc.load_gather(tbuf, [rows, dhi])
                    new = [(dvec + 1) & (_D - 1), accs[0] + vc * vt]
                    for j in range(_K):
                        vn = plsc.load_gather(nbuf, [rows + (j * C), dhi])
                        new.append(accs[1 + j] + vc * vn)
                    return tuple(new)

                init = (lanes,) + tuple(jnp.zeros((_L,), jnp.float32)
                                        for _ in range(_K + 1))
                accs = lax.fori_loop(0, _D, dbody, init)[1:]
                so = off + g * _L
                scores[pl.ds(so, _L)] = accs[0]
                for j in range(_K):
                    scores[pl.ds((1 + j) * PW + so, _L)] = -accs[1 + j]

        out_descs = [
            pltpu.async_copy(scores.at[pl.ds(r * PW, PW)],
                             out_hbm.at[pl.ds(r * B + wbase, PW)], sem)
            for r in range(_K + 1)
        ]
        for dsc in out_descs:
            dsc.wait()

    return k(center, context, negs.T, comb)


def _tc_relayout(tinT, toutT, *, V, W):
    def body(a_ref, b_ref, o_ref):
        o_ref[:, :_D] = a_ref[...].T
        o_ref[:, _D:] = b_ref[...].T

    return pl.pallas_call(
        body,
        grid=(pl.cdiv(V, W),),
        in_specs=[
            pl.BlockSpec((_D, W), lambda i: (0, i)),
            pl.BlockSpec((_D, W), lambda i: (0, i)),
        ],
        out_specs=pl.BlockSpec((W, 2 * _D), lambda i: (i, 0)),
        out_shape=jax.ShapeDtypeStruct((V, 2 * _D), jnp.float32),
    )(tinT, toutT)


def _tc_loss(x, *, B):
    def body(x_ref, o_ref):
        v = x_ref[...]
        ls = jnp.where(v < 0, v, 0.0) - jnp.log1p(jnp.exp(-jnp.abs(v)))
        o_ref[0, 0] = -jnp.sum(ls) / B

    return pl.pallas_call(
        body,
        out_shape=jax.ShapeDtypeStruct((1, 1), jnp.float32),
        out_specs=pl.BlockSpec(memory_space=pltpu.SMEM),
    )(x)


def kernel(center, context, negatives, input_embeddings, output_embeddings):
    B = center.shape[0]
    V = input_embeddings.shape[0]
    comb = _tc_relayout(input_embeddings.T, output_embeddings.T,
                        V=V, W=16384)
    scores = _sc_scores(center, context, negatives, comb, B=B, C=32)
    return _tc_loss(scores, B=B)[0, 0]


# speedup vs baseline: 1.0079x; 1.0079x over previous
import functools

import jax
import jax.numpy as jnp
from jax import lax
from jax.experimental import pallas as pl
from jax.experimental.pallas import tpu as pltpu
from jax.experimental.pallas import tpu_sc as plsc

_D = 64
_K = 20
_L = 16
_NC, _NS = 2, 16
_NW = _NC * _NS


def _sc_scores(center, context, negs, comb, *, B, C):
    PW = B // _NW
    NCH = PW // C
    mesh = plsc.VectorSubcoreMesh(core_axis_name="c", subcore_axis_name="s")

    @functools.partial(
        pl.kernel,
        out_type=jax.ShapeDtypeStruct(((_K + 1) * B,), jnp.float32),
        mesh=mesh,
        compiler_params=pltpu.CompilerParams(needs_layout_passes=False),
        scratch_types=[
            pltpu.VMEM((PW,), jnp.int32),
            pltpu.VMEM((PW,), jnp.int32),
            pltpu.VMEM((_K * PW,), jnp.int32),
            pltpu.VMEM((C, 2 * _D), jnp.float32),
            pltpu.VMEM((C, 2 * _D), jnp.float32),
            pltpu.VMEM((_K * C, 2 * _D), jnp.float32),
            pltpu.VMEM(((_K + 1) * PW,), jnp.float32),
            pltpu.SemaphoreType.DMA,
        ],
    )
    def k(center_hbm, context_hbm, negsT_hbm, comb_hbm, out_hbm,
          cidx, tidx, nidx, cbuf, tbuf, nbuf, scores, sem):
        wid = lax.axis_index("s") * _NC + lax.axis_index("c")
        wbase = wid * PW
        idx_descs = [
            pltpu.async_copy(center_hbm.at[pl.ds(wbase, PW)], cidx, sem),
            pltpu.async_copy(context_hbm.at[pl.ds(wbase, PW)], tidx, sem),
        ]
        for j in range(_K):
            idx_descs.append(
                pltpu.async_copy(negsT_hbm.at[j, pl.ds(wbase, PW)],
                                 nidx.at[pl.ds(j * PW, PW)], sem))
        for dsc in idx_descs:
            dsc.wait()

        for ch in range(NCH):
            off = ch * C
            descs = [
                pltpu.async_copy(comb_hbm.at[cidx.at[pl.ds(off, C)]],
                                 cbuf, sem),
                pltpu.async_copy(comb_hbm.at[tidx.at[pl.ds(off, C)]],
                                 tbuf, sem),
            ]
            for j in range(_K):
                descs.append(
                    pltpu.async_copy(
                        comb_hbm.at[nidx.at[pl.ds(j * PW + off, C)]],
                        nbuf.at[pl.ds(j * C, C), :], sem))
            for dsc in descs:
                dsc.wait()

            lanes = lax.iota(jnp.int32, _L)
            for g in range(C // _L):
                rows = lanes + (g * _L)

                def dbody(i, carry, rows=rows):
                    dvec = carry[0]
                    accs = carry[1:]
                    dhi = dvec + _D
                    vc = plsc.load_gather(cbuf, [rows, dvec])
                    vt = plsc.load_gather(tbuf, [rows, dhi])
                    new = [(dvec + 1) & (_D - 1), accs[0] + vc * vt]
                    for j in range(_K):
                        vn = plsc.load_gather(nbuf, [rows + (j * C), dhi])
                        new.append(accs[1 + j] + vc * vn)
                    return tuple(new)

                init = (lanes,) + tuple(jnp.zeros((_L,), jnp.float32)
                                        for _ in range(_K + 1))
                accs = lax.fori_loop(0, _D, dbody, init)[1:]
                so = off + g * _L
                scores[pl.ds(so, _L)] = accs[0]
                for j in range(_K):
                    scores[pl.ds((1 + j) * PW + so, _L)] = -accs[1 + j]

        out_descs = [
            pltpu.async_copy(scores.at[pl.ds(r * PW, PW)],
                             out_hbm.at[pl.ds(r * B + wbase, PW)], sem)
            for r in range(_K + 1)
        ]
        for dsc in out_descs:
            dsc.wait()

    return k(center, context, negs.T, comb)


def _tc_relayout(tinT, toutT, *, V, W):
    def body(a_ref, b_ref, o_ref):
        o_ref[:, :_D] = a_ref[...].T
        o_ref[:, _D:] = b_ref[...].T

    return pl.pallas_call(
        body,
        grid=(pl.cdiv(V, W),),
        in_specs=[
            pl.BlockSpec((_D, W), lambda i: (0, i)),
            pl.BlockSpec((_D, W), lambda i: (0, i)),
        ],
        out_specs=pl.BlockSpec((W, 2 * _D), lambda i: (i, 0)),
        out_shape=jax.ShapeDtypeStruct((V, 2 * _D), jnp.float32),
    )(tinT, toutT)


def _tc_loss(x, *, B):
    def body(x_ref, o_ref):
        v = x_ref[...]
        ls = jnp.where(v < 0, v, 0.0) - jnp.log1p(jnp.exp(-jnp.abs(v)))
        o_ref[0, 0] = -jnp.sum(ls) / B

    return pl.pallas_call(
        body,
        out_shape=jax.ShapeDtypeStruct((1, 1), jnp.float32),
        out_specs=pl.BlockSpec(memory_space=pltpu.SMEM),
    )(x)


def kernel(center, context, negatives, input_embeddings, output_embeddings):
    B = center.shape[0]
    V = input_embeddings.shape[0]
    comb = _tc_relayout(input_embeddings.T, output_embeddings.T,
                        V=V, W=20480)
    scores = _sc_scores(center, context, negatives, comb, B=B, C=32)
    return _tc_loss(scores, B=B)[0, 0]
